# baseline (speedup 1.0000x reference)
import functools

import jax
import jax.numpy as jnp
import numpy as np
from jax import lax
from jax.experimental import pallas as pl
from jax.experimental.pallas import tpu as pltpu
from jax.experimental.pallas import tpu_sc as plsc
from jax._src import config as _jax_config

N_VALUES = 10
BATCH = 16384
DIM = 17
V_ROWS = 100000
WT_ROWS = V_ROWS * DIM // 8
ROWS_PER_COL = V_ROWS // 8
NC = 2
NS = 16
NW = NC * NS
B_PER_W = BATCH // NW
N_GROUPS = B_PER_W // 16
N_WAVES = 4
GROUPS_PER_WAVE = N_GROUPS // N_WAVES

_mesh = plsc.VectorSubcoreMesh(core_axis_name="c", subcore_axis_name="s")


def _sc_lookup_body(
    xt_hbm, wt_hbm, out_hbm, z1_hbm, z2_hbm,
    x_v, idx_v, o_v, buf_v, out_v, z_v, sems,
):
    wid = lax.axis_index("s") * NC + lax.axis_index("c")
    base = wid * B_PER_W
    pltpu.sync_copy(xt_hbm.at[:, pl.ds(base, B_PER_W)], x_v)

    lanes = lax.iota(jnp.int32, 16)
    zero16 = jnp.zeros((16,), jnp.float32)
    for i in range(N_GROUPS):
        s16 = pl.ds(i * 16, 16)
        acc = x_v[0, s16]
        for j in range(1, 5):
            acc = acc * N_VALUES + x_v[j, s16]
        o_v[s16] = acc & 7
        hi = acc >> 3
        for c in range(DIM):
            idx_v[4 * c + i // 8, pl.ds((i % 8) * 16, 16)] = (
                hi + c * ROWS_PER_COL
            )

    copies = [
        [
            pltpu.async_copy(
                wt_hbm.at[idx_v.at[jnp.int32(4 * c + w)]],
                buf_v.at[pl.ds((4 * c + w) * 128, 128)],
                sems.at[w],
            )
            for c in range(DIM)
        ]
        for w in range(N_WAVES)
    ]

    for w in range(N_WAVES):
        for cp in copies[w]:
            cp.wait()

        @pl.loop(
            np.int32(w * GROUPS_PER_WAVE * 16),
            np.int32((w + 1) * GROUPS_PER_WAVE * 16),
            step=np.int32(16),
        )
        def extract_group(b):
            r16 = b + lanes
            o16 = o_v[pl.ds(b, 16)]
            for c in range(DIM):
                row16 = r16 + c * B_PER_W
                v = plsc.load_gather(buf_v, [row16, o16])
                out_v[c, pl.ds(b, 16)] = v.astype(jnp.int32) + 1

    for i in range(N_GROUPS):
        z_v[pl.ds(i * 16, 16)] = zero16

    pltpu.sync_copy(out_v, out_hbm.at[:, pl.ds(base, B_PER_W)])
    for zo in (z1_hbm, z2_hbm):
        for c in range(DIM):
            pltpu.sync_copy(z_v, zo.at[c, pl.ds(base, B_PER_W)])


_sc_lookup = functools.partial(
    pl.kernel,
    mesh=_mesh,
    out_type=(
        jax.ShapeDtypeStruct((DIM, BATCH), jnp.int32),
        jax.ShapeDtypeStruct((DIM, BATCH), jnp.float32),
        jax.ShapeDtypeStruct((DIM, BATCH), jnp.float32),
    ),
    scratch_types=[
        pltpu.VMEM((5, B_PER_W), jnp.int32),
        pltpu.VMEM((DIM * 4, 128), jnp.int32),
        pltpu.VMEM((B_PER_W,), jnp.int32),
        pltpu.VMEM((DIM * B_PER_W, 8), jnp.float32),
        pltpu.VMEM((DIM, B_PER_W), jnp.int32),
        pltpu.VMEM((B_PER_W,), jnp.float32),
        pltpu.SemaphoreType.DMA((N_WAVES,)),
    ],
    compiler_params=pltpu.CompilerParams(
        use_tc_tiling_on_sc=False, needs_layout_passes=False
    ),
)(_sc_lookup_body)


def kernel(x, W):
    xw = x.astype(jnp.int32).T
    wt = W.T.reshape(WT_ROWS, 8)
    with _jax_config.enable_x64(False):
        g32t, z1t, z2t = _sc_lookup(xw, wt)
    g = g32t.T.astype(jnp.int64)
    return (g, z1t.T, z2t.T)


# speedup vs baseline: 1.0586x; 1.0586x over previous
import functools

import jax
import jax.numpy as jnp
import numpy as np
from jax import lax
from jax.experimental import pallas as pl
from jax.experimental.pallas import tpu as pltpu
from jax.experimental.pallas import tpu_sc as plsc
from jax._src import config as _jax_config

N_VALUES = 10
BATCH = 16384
DIM = 17
V_ROWS = 100000
WT_ROWS = V_ROWS * DIM // 8
ROWS_PER_COL = V_ROWS // 8
NC = 2
NS = 16
NW = NC * NS
B_PER_W = BATCH // NW
N_GROUPS = B_PER_W // 16
N_CHUNKS = DIM * B_PER_W // 128

_mesh = plsc.VectorSubcoreMesh(core_axis_name="c", subcore_axis_name="s")


def _sc_lookup_body(xt_hbm, wt_hbm, out_hbm, x_v, idx_v, o_v, buf_v, out_v, sem):
    wid = lax.axis_index("s") * NC + lax.axis_index("c")
    base = wid * B_PER_W
    pltpu.sync_copy(xt_hbm.at[:, pl.ds(base, B_PER_W)], x_v)

    lanes = lax.iota(jnp.int32, 16)
    for i in range(N_GROUPS):
        s16 = pl.ds(i * 16, 16)
        acc = x_v[0, s16]
        for j in range(1, 5):
            acc = acc * N_VALUES + x_v[j, s16]
        o_v[s16] = acc & 7
        hi = acc >> 3
        for c in range(DIM):
            idx_v[4 * c + i // 8, pl.ds((i % 8) * 16, 16)] = (
                hi + c * ROWS_PER_COL
            )

    copies = [
        pltpu.async_copy(
            wt_hbm.at[idx_v.at[jnp.int32(j)]],
            buf_v.at[pl.ds(j * 128, 128)],
            sem,
        )
        for j in range(N_CHUNKS)
    ]
    for c in copies:
        c.wait()

    @pl.loop(np.int32(0), np.int32(B_PER_W), step=np.int32(16))
    def extract_group(b):
        r16 = b + lanes
        o16 = o_v[pl.ds(b, 16)]
        for c in range(DIM):
            row16 = r16 + c * B_PER_W
            v = plsc.load_gather(buf_v, [row16, o16])
            out_v[c, pl.ds(b, 16)] = v.astype(jnp.int32) + 1

    pltpu.sync_copy(out_v, out_hbm.at[:, pl.ds(base, B_PER_W)])


_sc_lookup = functools.partial(
    pl.kernel,
    mesh=_mesh,
    out_type=jax.ShapeDtypeStruct((DIM, BATCH), jnp.int32),
    scratch_types=[
        pltpu.VMEM((5, B_PER_W), jnp.int32),
        pltpu.VMEM((N_CHUNKS, 128), jnp.int32),
        pltpu.VMEM((B_PER_W,), jnp.int32),
        pltpu.VMEM((DIM * B_PER_W, 8), jnp.float32),
        pltpu.VMEM((DIM, B_PER_W), jnp.int32),
        pltpu.SemaphoreType.DMA,
    ],
    compiler_params=pltpu.CompilerParams(
        use_tc_tiling_on_sc=False, needs_layout_passes=False
    ),
)(_sc_lookup_body)


def kernel(x, W):
    xw = x.astype(jnp.int32).T
    wt = W.T.reshape(WT_ROWS, 8)
    with _jax_config.enable_x64(False):
        g32t = _sc_lookup(xw, wt)
    g = g32t.T.astype(jnp.int64)
    zeros = jnp.zeros((x.shape[0], W.shape[1]), dtype=jnp.float32)
    return (g, zeros, zeros)


# speedup vs baseline: 1.0913x; 1.0309x over previous
import functools

import jax
import jax.numpy as jnp
import numpy as np
from jax import lax
from jax.experimental import pallas as pl
from jax.experimental.pallas import tpu as pltpu
from jax.experimental.pallas import tpu_sc as plsc
from jax._src import config as _jax_config

N_VALUES = 10
BATCH = 16384
DIM = 17
V_ROWS = 100000
WT_ROWS = V_ROWS * DIM // 8
ROWS_PER_COL = V_ROWS // 8
NC = 2
NS = 16
NW = NC * NS
B_PER_W = BATCH // NW
N_GROUPS = B_PER_W // 16
N_CHUNKS = DIM * B_PER_W // 128

_mesh = plsc.VectorSubcoreMesh(core_axis_name="c", subcore_axis_name="s")


def _sc_lookup_body(xt_hbm, wt_hbm, out_hbm, x_v, idx_v, o_v, buf_v, out_v, sem):
    wid = lax.axis_index("s") * NC + lax.axis_index("c")
    base = wid * B_PER_W
    pltpu.sync_copy(xt_hbm.at[:, pl.ds(base, B_PER_W)], x_v)

    lanes = lax.iota(jnp.int32, 16)
    for i in range(N_GROUPS):
        s16 = pl.ds(i * 16, 16)
        acc = x_v[0, s16]
        for j in range(1, 5):
            acc = acc * N_VALUES + x_v[j, s16]
        o_v[s16] = acc & 7
        hi = acc >> 3
        for c in range(DIM):
            idx_v[4 * c + i // 8, pl.ds((i % 8) * 16, 16)] = (
                hi + c * ROWS_PER_COL
            )

    waves = [
        [
            pltpu.async_copy(
                wt_hbm.at[idx_v.at[jnp.int32(j)]],
                buf_v.at[pl.ds(j * 128, 128)],
                sem.at[w],
            )
            for j in range(N_CHUNKS)
            if (j % 4) // 2 == w
        ]
        for w in range(2)
    ]

    for w in range(2):
        for cp in waves[w]:
            cp.wait()

        @pl.loop(
            np.int32(w * B_PER_W // 2),
            np.int32((w + 1) * B_PER_W // 2),
            step=np.int32(16),
        )
        def extract_group(b):
            r16 = b + lanes
            o16 = o_v[pl.ds(b, 16)]
            for c in range(DIM):
                row16 = r16 + c * B_PER_W
                v = plsc.load_gather(buf_v, [row16, o16])
                out_v[c, pl.ds(b, 16)] = v.astype(jnp.int32) + 1

    pltpu.sync_copy(out_v, out_hbm.at[:, pl.ds(base, B_PER_W)])


_sc_lookup = functools.partial(
    pl.kernel,
    mesh=_mesh,
    out_type=jax.ShapeDtypeStruct((DIM, BATCH), jnp.int32),
    scratch_types=[
        pltpu.VMEM((5, B_PER_W), jnp.int32),
        pltpu.VMEM((N_CHUNKS, 128), jnp.int32),
        pltpu.VMEM((B_PER_W,), jnp.int32),
        pltpu.VMEM((DIM * B_PER_W, 8), jnp.float32),
        pltpu.VMEM((DIM, B_PER_W), jnp.int32),
        pltpu.SemaphoreType.DMA((2,)),
    ],
    compiler_params=pltpu.CompilerParams(
        use_tc_tiling_on_sc=False, needs_layout_passes=False
    ),
)(_sc_lookup_body)


def kernel(x, W):
    xw = x.astype(jnp.int32).T
    wt = W.T.reshape(WT_ROWS, 8)
    with _jax_config.enable_x64(False):
        g32t = _sc_lookup(xw, wt)
    g = g32t.T.astype(jnp.int64)
    zeros = jnp.zeros((x.shape[0], W.shape[1]), dtype=jnp.float32)
    return (g, zeros, zeros)
